# baseline (speedup 1.0000x reference)
import jax
import jax.numpy as jnp
from jax.experimental import pallas as pl
from jax.experimental.pallas import tpu as pltpu

_TM = 512
_TE = 1024
_TC = 256


def _moe_kernel(x_ref, wg_ref, wu_ref, wd_ref, o_ref, acc_ref, scale_ref):
    j = pl.program_id(1)
    nj = pl.num_programs(1)

    @pl.when(j == 0)
    def _gate():
        logits = jax.lax.dot_general(
            x_ref[...], wg_ref[...],
            (((1,), (1,)), ((), ())),
            preferred_element_type=jnp.float32,
        )
        m = jnp.max(logits, axis=1, keepdims=True)
        e = jnp.exp(logits - m)
        s = jnp.sum(e, axis=1)
        i1 = jnp.argmax(logits, axis=1)
        one_hot = jax.lax.broadcasted_iota(jnp.int32, logits.shape, 1) == i1[:, None]
        e1 = jnp.max(e, axis=1)
        e2 = jnp.max(jnp.where(one_hot, -jnp.inf, e), axis=1)
        v = (e1 + e2) / s
        scale_ref[...] = (v / (v + 1e-9))[:, None]

    x_blk = x_ref[...]
    part = None
    for c in range(0, _TE, _TC):
        h = jax.lax.dot_general(
            x_blk, wu_ref[c:c + _TC, :],
            (((1,), (1,)), ((), ())),
            preferred_element_type=jnp.float32,
        )
        h = h * jax.nn.sigmoid(h)
        p = jax.lax.dot_general(
            h, wd_ref[:, c:c + _TC],
            (((1,), (1,)), ((), ())),
            preferred_element_type=jnp.float32,
        )
        part = p if part is None else part + p

    @pl.when(j == 0)
    def _init():
        acc_ref[...] = part

    @pl.when(j > 0)
    def _acc():
        acc_ref[...] += part

    @pl.when(j == nj - 1)
    def _flush():
        o_ref[...] = acc_ref[...] * scale_ref[...]


def kernel(x, W_gate, W_up, W_down):
    B, S, D = x.shape
    T = B * S
    ED = W_up.shape[0]
    xf = x.reshape(T, D)
    grid = (T // _TM, ED // _TE)
    out = pl.pallas_call(
        _moe_kernel,
        grid=grid,
        in_specs=[
            pl.BlockSpec((_TM, D), lambda i, j: (i, 0)),
            pl.BlockSpec(W_gate.shape, lambda i, j: (0, 0)),
            pl.BlockSpec((_TE, D), lambda i, j: (j, 0)),
            pl.BlockSpec((D, _TE), lambda i, j: (0, j)),
        ],
        out_specs=pl.BlockSpec((_TM, D), lambda i, j: (i, 0)),
        out_shape=jax.ShapeDtypeStruct((T, D), jnp.float32),
        scratch_shapes=[
            pltpu.VMEM((_TM, D), jnp.float32),
            pltpu.VMEM((_TM, 1), jnp.float32),
        ],
        compiler_params=pltpu.CompilerParams(
            dimension_semantics=("parallel", "arbitrary"),
        ),
    )(xf, W_gate, W_up, W_down)
    return out.reshape(B, S, D)


# speedup vs baseline: 1.1218x; 1.1218x over previous
import jax
import jax.numpy as jnp
from jax.experimental import pallas as pl
from jax.experimental.pallas import tpu as pltpu

_TM = 512
_TE = 2048


def _moe_kernel(x_ref, wg_ref, wu_ref, wd_ref, o_ref, acc_ref, scale_ref):
    j = pl.program_id(1)
    nj = pl.num_programs(1)

    @pl.when(j == 0)
    def _gate():
        logits = jax.lax.dot_general(
            x_ref[...], wg_ref[...],
            (((1,), (1,)), ((), ())),
            preferred_element_type=jnp.float32,
        )
        m = jnp.max(logits, axis=1, keepdims=True)
        e = jnp.exp(logits - m)
        s = jnp.sum(e, axis=1)
        i1 = jnp.argmax(logits, axis=1)
        one_hot = jax.lax.broadcasted_iota(jnp.int32, logits.shape, 1) == i1[:, None]
        e1 = jnp.max(e, axis=1)
        e2 = jnp.max(jnp.where(one_hot, -jnp.inf, e), axis=1)
        v = (e1 + e2) / s
        scale_ref[...] = (v / (v + 1e-9))[:, None]

    h = jax.lax.dot_general(
        x_ref[...], wu_ref[...],
        (((1,), (1,)), ((), ())),
        preferred_element_type=jnp.float32,
    )
    h = (h * jax.nn.sigmoid(h)).astype(jnp.bfloat16)
    part = jax.lax.dot_general(
        h, wd_ref[...],
        (((1,), (1,)), ((), ())),
        preferred_element_type=jnp.float32,
    )

    @pl.when(j == 0)
    def _init():
        acc_ref[...] = part

    @pl.when(j > 0)
    def _acc():
        acc_ref[...] += part

    @pl.when(j == nj - 1)
    def _flush():
        o_ref[...] = acc_ref[...] * scale_ref[...]


def kernel(x, W_gate, W_up, W_down):
    B, S, D = x.shape
    T = B * S
    ED = W_up.shape[0]
    xf = x.reshape(T, D).astype(jnp.bfloat16)
    wg = W_gate.astype(jnp.bfloat16)
    wu = W_up.astype(jnp.bfloat16)
    wd = W_down.astype(jnp.bfloat16)
    grid = (T // _TM, ED // _TE)
    out = pl.pallas_call(
        _moe_kernel,
        grid=grid,
        in_specs=[
            pl.BlockSpec((_TM, D), lambda i, j: (i, 0)),
            pl.BlockSpec(W_gate.shape, lambda i, j: (0, 0)),
            pl.BlockSpec((_TE, D), lambda i, j: (j, 0)),
            pl.BlockSpec((D, _TE), lambda i, j: (0, j)),
        ],
        out_specs=pl.BlockSpec((_TM, D), lambda i, j: (i, 0)),
        out_shape=jax.ShapeDtypeStruct((T, D), jnp.float32),
        scratch_shapes=[
            pltpu.VMEM((_TM, D), jnp.float32),
            pltpu.VMEM((_TM, 1), jnp.float32),
        ],
        compiler_params=pltpu.CompilerParams(
            dimension_semantics=("parallel", "arbitrary"),
        ),
    )(xf, wg, wu, wd)
    return out.reshape(B, S, D)
